# Initial kernel scaffold; baseline (speedup 1.0000x reference)
import jax
import jax.numpy as jnp
from jax.experimental import pallas as pl


def kernel(x, edge_index, W_e1, b_e1, W_e2, b_e2, W_a1, b_a1, W_a2, b_a2, W_s1, b_s1):
    raise NotImplementedError("write your pallas kernel here")


# speedup vs baseline: 6.4261x; 6.4261x over previous
import functools

import jax
import jax.numpy as jnp
from jax import lax
from jax.experimental import pallas as pl
from jax.experimental.pallas import tpu as pltpu
from jax.experimental.pallas import tpu_sc as plsc

N = 10000
NFEAT = 128
NHID = 64
E = 320000

NC = 2
NS = 16
NW = NC * NS
CHUNK = 128
NCH_W = 80
EPAD = NW * NCH_W * CHUNK
NPAD = 10240
RPT = NPAD // NS

_MESH = plsc.VectorSubcoreMesh(core_axis_name="c", subcore_axis_name="s")


def _make_prop(D):
    @functools.partial(
        pl.kernel,
        mesh=_MESH,
        out_type=jax.ShapeDtypeStruct((NC, NPAD, D), jnp.float32),
        scratch_types=[
            pltpu.VMEM((NCH_W, CHUNK), jnp.int32),
            pltpu.VMEM((NCH_W, CHUNK), jnp.int32),
            pltpu.VMEM((CHUNK, D), jnp.float32),
            pltpu.VMEM_SHARED((NPAD, D), jnp.float32),
            pltpu.SemaphoreType.DMA,
        ],
    )
    def prop(g_hbm, src_hbm, dst_hbm, out_hbm, srcv, dstv, rows, acc, sem):
        c = lax.axis_index("c")
        s = lax.axis_index("s")
        wid = c * NS + s
        zero16 = jnp.zeros((16,), jnp.float32)

        def zrow(i, _):
            for cc in range(D // 16):
                rows[i, pl.ds(cc * 16, 16)] = zero16
            return 0

        lax.fori_loop(0, CHUNK, zrow, 0)
        base = s * RPT

        def zacc(k, _):
            pltpu.sync_copy(rows, acc.at[pl.ds(base + k * CHUNK, CHUNK)])
            return 0

        lax.fori_loop(0, RPT // CHUNK, zacc, 0)
        pltpu.sync_copy(src_hbm.at[wid], srcv)
        pltpu.sync_copy(dst_hbm.at[wid], dstv)
        plsc.subcore_barrier()

        def body(j, _):
            pltpu.async_copy(g_hbm.at[srcv.at[j]], rows, sem).wait()
            pltpu.sync_copy(rows, acc.at[dstv.at[j]], add=True)
            return 0

        lax.fori_loop(0, NCH_W, body, 0)
        plsc.subcore_barrier()
        pltpu.sync_copy(acc.at[pl.ds(base, RPT)],
                        out_hbm.at[c, pl.ds(base, RPT)])

    return prop


DDEG = 16


@functools.partial(
    pl.kernel,
    mesh=_MESH,
    out_type=jax.ShapeDtypeStruct((NC, NPAD, DDEG), jnp.float32),
    scratch_types=[
        pltpu.VMEM((NCH_W, CHUNK), jnp.int32),
        pltpu.VMEM((CHUNK, DDEG), jnp.float32),
        pltpu.VMEM_SHARED((NPAD, DDEG), jnp.float32),
    ],
)
def _deg_kernel(dst_hbm, out_hbm, dstv, ones, acc):
    c = lax.axis_index("c")
    s = lax.axis_index("s")
    wid = c * NS + s
    zero16 = jnp.zeros((16,), jnp.float32)

    def zrow(i, _):
        ones[i, pl.ds(0, 16)] = zero16
        return 0

    lax.fori_loop(0, CHUNK, zrow, 0)
    base = s * RPT

    def zacc(k, _):
        pltpu.sync_copy(ones, acc.at[pl.ds(base + k * CHUNK, CHUNK)])
        return 0

    lax.fori_loop(0, RPT // CHUNK, zacc, 0)

    one16 = jnp.ones((16,), jnp.float32)

    def orow(i, _):
        ones[i, pl.ds(0, 16)] = one16
        return 0

    lax.fori_loop(0, CHUNK, orow, 0)
    pltpu.sync_copy(dst_hbm.at[wid], dstv)
    plsc.subcore_barrier()

    def body(j, _):
        pltpu.sync_copy(ones, acc.at[dstv.at[j]], add=True)
        return 0

    lax.fori_loop(0, NCH_W, body, 0)
    plsc.subcore_barrier()
    pltpu.sync_copy(acc.at[pl.ds(base, RPT)], out_hbm.at[c, pl.ds(base, RPT)])


RB = 1024
NB = NPAD // RB


def _dinv(d0, d1):
    return lax.rsqrt(d0[:, :1] + d1[:, :1] + 1.0)


def _k1_body(x_ref, w_ref, d0, d1, out_ref):
    out_ref[...] = _dinv(d0, d1) * jnp.dot(
        x_ref[...], w_ref[...], preferred_element_type=jnp.float32)


def _k2_body(t0, t1, g, w, b, d0, d1, out_ref):
    dinv = _dinv(d0, d1)
    h = jnp.maximum(dinv * (t0[...] + t1[...] + g[...]) + b[...], 0.0)
    out_ref[...] = dinv * jnp.dot(h, w[...], preferred_element_type=jnp.float32)


def _k4_body(t0, t1, g, ba1, bs1, d0, d1, g4_ref, s_ref):
    dinv = _dinv(d0, d1)
    pa = dinv * (t0[:, :NHID] + t1[:, :NHID] + g[:, :NHID])
    xa = dinv * jnp.maximum(pa + ba1[...], 0.0)
    g4_ref[...] = jnp.concatenate(
        [xa, jnp.zeros_like(xa)], axis=1)
    ps = dinv * (t0[:, NHID:] + t1[:, NHID:] + g[:, NHID:])
    s_ref[...] = jnp.maximum(ps + bs1[...], 0.0)


def _k5_body(t0, t1, g, w, b, d0, d1, out_ref):
    dinv = _dinv(d0, d1)
    p = dinv * (t0[...] + t1[...] + g[...])
    out_ref[...] = jnp.maximum(
        jnp.dot(p, w[...], preferred_element_type=jnp.float32) + b[...], 0.0)


def _mm_body(a_ref, b_ref, o_ref):
    o_ref[...] = lax.dot_general(
        a_ref[...], b_ref[...], (((1,), (1,)), ((), ())),
        preferred_element_type=jnp.float32)


def _row_spec(w):
    return pl.BlockSpec((RB, w), lambda i: (i, 0))


def _full_spec(shape):
    return pl.BlockSpec(shape, lambda i: tuple(0 for _ in shape))


def _tc_stage(body, n_out, out_w, ins, widths, fulls):
    in_specs = []
    for a, w, f in zip(ins, widths, fulls):
        in_specs.append(_full_spec(a.shape) if f else _row_spec(w))
    if n_out == 1:
        out_shape = jax.ShapeDtypeStruct((NPAD, out_w[0]), jnp.float32)
        out_specs = _row_spec(out_w[0])
    else:
        out_shape = tuple(
            jax.ShapeDtypeStruct((NPAD, w), jnp.float32) for w in out_w)
        out_specs = tuple(_row_spec(w) for w in out_w)
    return pl.pallas_call(
        body, grid=(NB,), in_specs=in_specs,
        out_specs=out_specs, out_shape=out_shape)(*ins)


MMB = 400


def _struct_mm(s):
    nb = N // MMB
    return pl.pallas_call(
        _mm_body,
        grid=(nb,),
        in_specs=[
            pl.BlockSpec((MMB, NHID), lambda i: (i, 0)),
            pl.BlockSpec((N, NHID), lambda i: (0, 0)),
        ],
        out_specs=pl.BlockSpec((MMB, N), lambda i: (i, 0)),
        out_shape=jax.ShapeDtypeStruct((N, N), jnp.float32),
    )(s, s)


DP = 128
_prop = _make_prop(DP)


def kernel(x, edge_index, W_e1, b_e1, W_e2, b_e2, W_a1, b_a1, W_a2, b_a2,
           W_s1, b_s1):
    pad = jnp.full((EPAD - E,), NPAD - 1, dtype=jnp.int32)
    srcp = jnp.concatenate([edge_index[0], pad]).reshape(NW, NCH_W, CHUNK)
    dstp = jnp.concatenate([edge_index[1], pad]).reshape(NW, NCH_W, CHUNK)
    x_pad = jnp.zeros((NPAD, NFEAT), jnp.float32).at[:N].set(x)

    deg = _deg_kernel(dstp)
    d0, d1 = deg[0], deg[1]

    zc = jnp.zeros((1, NHID), jnp.float32)
    zw = jnp.zeros((NHID, DP), jnp.float32)
    W1p = jnp.concatenate([W_e1, jnp.zeros((NFEAT, NHID), jnp.float32)],
                          axis=1)
    W2p = jnp.concatenate(
        [jnp.concatenate([W_e2, jnp.zeros((NHID, NHID), jnp.float32)],
                         axis=1), zw], axis=0)
    Wap = jnp.concatenate(
        [jnp.concatenate([W_a1, W_s1], axis=1), zw], axis=0)
    Wa2p = jnp.concatenate([W_a2, jnp.zeros((NHID, NFEAT), jnp.float32)],
                           axis=0)
    b_e1r = jnp.concatenate([b_e1.reshape(1, NHID), zc], axis=1)
    b_e2r = jnp.concatenate([b_e2.reshape(1, NHID), zc], axis=1)
    b_a1r = b_a1.reshape(1, NHID)
    b_a2r = b_a2.reshape(1, NFEAT)
    b_s1r = b_s1.reshape(1, NHID)

    g1 = _tc_stage(_k1_body, 1, (DP,),
                   (x_pad, W1p, d0, d1),
                   (NFEAT, None, DDEG, DDEG),
                   (False, True, False, False))
    t1 = _prop(g1, srcp, dstp)
    g2 = _tc_stage(_k2_body, 1, (DP,),
                   (t1[0], t1[1], g1, W2p, b_e1r, d0, d1),
                   (DP, DP, DP, None, None, DDEG, DDEG),
                   (False, False, False, True, True, False, False))
    t2 = _prop(g2, srcp, dstp)
    gas = _tc_stage(_k2_body, 1, (DP,),
                    (t2[0], t2[1], g2, Wap, b_e2r, d0, d1),
                    (DP, DP, DP, None, None, DDEG, DDEG),
                    (False, False, False, True, True, False, False))
    t3 = _prop(gas, srcp, dstp)
    g4, s_full = _tc_stage(_k4_body, 2, (DP, NHID),
                           (t3[0], t3[1], gas, b_a1r, b_s1r, d0, d1),
                           (DP, DP, DP, None, None, DDEG, DDEG),
                           (False, False, False, True, True, False, False))
    t4 = _prop(g4, srcp, dstp)
    xh_full = _tc_stage(_k5_body, 1, (NFEAT,),
                        (t4[0], t4[1], g4, Wa2p, b_a2r, d0, d1),
                        (DP, DP, DP, None, None, DDEG, DDEG),
                        (False, False, False, True, True, False, False))
    s = s_full[:N]
    A_hat = _struct_mm(s)
    return (A_hat, xh_full[:N])
